# Initial kernel scaffold; baseline (speedup 1.0000x reference)
import jax
import jax.numpy as jnp
from jax.experimental import pallas as pl


def kernel(x, edge_index, W1, b1, W2, b2, MW1, Mb1, MW2, Mb2):
    raise NotImplementedError("write your pallas kernel here")


# speedup vs baseline: 2.9326x; 2.9326x over previous
import functools

import jax
import jax.numpy as jnp
from jax import lax
from jax.experimental import pallas as pl
from jax.experimental.pallas import tpu as pltpu
from jax.experimental.pallas import tpu_sc as plsc

N = 10000
NP = 10240
E = 160000
EP = 163840
C = 256
H = 128
NC = 2
NS = 16
CH = 128
ROWS_T = NP // NS

_F32 = jnp.float32
_HIGH = jax.lax.Precision.HIGHEST


def _mesh():
    return plsc.VectorSubcoreMesh(core_axis_name="c", subcore_axis_name="s")


def _deg_body(dst_hbm, zeros_hbm, ones_hbm, out_hbm, acc_sh, idx_v, ones_v, sem):
    del sem
    c = lax.axis_index("c")
    s = lax.axis_index("s")
    pltpu.sync_copy(zeros_hbm, acc_sh.at[pl.ds(s * ROWS_T, ROWS_T)])
    pltpu.sync_copy(ones_hbm, ones_v)
    plsc.subcore_barrier()
    base_e = (c * NS + s) * (EP // (NC * NS))

    def chunk(i, carry):
        pltpu.sync_copy(dst_hbm.at[pl.ds(base_e + i * CH, CH)], idx_v)
        pltpu.sync_copy(ones_v, acc_sh.at[idx_v], add=True)
        return carry

    lax.fori_loop(0, EP // (NC * NS) // CH, chunk, 0)
    plsc.subcore_barrier()
    pltpu.sync_copy(acc_sh.at[pl.ds(s * ROWS_T, ROWS_T)],
                    out_hbm.at[pl.ds(c * NP + s * ROWS_T, ROWS_T)])


def _deg(dst, zeros_t, ones_c):
    return pl.kernel(
        _deg_body,
        out_type=jax.ShapeDtypeStruct((NC * NP,), _F32),
        mesh=_mesh(),
        compiler_params=pltpu.CompilerParams(needs_layout_passes=False),
        scratch_types=[
            pltpu.VMEM_SHARED((NP,), _F32),
            pltpu.VMEM((CH,), jnp.int32),
            pltpu.VMEM((CH,), _F32),
            pltpu.SemaphoreType.DMA,
        ],
    )(dst, zeros_t, ones_c)


def _conv_body(hs_cat_hbm, src_hbm, dst_hbm, zrows_hbm, out_hbm,
               acc_sh, si_v, so_v, di_v, rows_v, sem):
    c = lax.axis_index("c")
    s = lax.axis_index("s")

    def z(i, carry):
        pltpu.sync_copy(zrows_hbm, acc_sh.at[pl.ds(s * ROWS_T + i * CH, CH)])
        return carry

    lax.fori_loop(0, ROWS_T // CH, z, 0)
    plsc.subcore_barrier()
    row_off = c * NP
    e_per_tile = EP // NS

    def chunk(i, carry):
        base = s * e_per_tile + i * CH
        pltpu.sync_copy(src_hbm.at[pl.ds(base, CH)], si_v)
        for g in range(CH // 16):
            so_v[pl.ds(g * 16, 16)] = si_v[pl.ds(g * 16, 16)] + row_off
        pltpu.async_copy(hs_cat_hbm.at[so_v], rows_v, sem).wait()
        pltpu.sync_copy(dst_hbm.at[pl.ds(base, CH)], di_v)
        pltpu.sync_copy(rows_v, acc_sh.at[di_v], add=True)
        return carry

    lax.fori_loop(0, e_per_tile // CH, chunk, 0)
    plsc.subcore_barrier()

    def co(i, carry):
        r = s * ROWS_T + i * CH
        pltpu.sync_copy(acc_sh.at[pl.ds(r, CH)],
                        out_hbm.at[pl.ds(c * NP + r, CH)])
        return carry

    lax.fori_loop(0, ROWS_T // CH, co, 0)


def _conv(hs_cat, src, dst, zrows):
    return pl.kernel(
        _conv_body,
        out_type=jax.ShapeDtypeStruct((NC * NP, H), _F32),
        mesh=_mesh(),
        compiler_params=pltpu.CompilerParams(needs_layout_passes=False),
        scratch_types=[
            pltpu.VMEM_SHARED((NP, H), _F32),
            pltpu.VMEM((CH,), jnp.int32),
            pltpu.VMEM((CH,), jnp.int32),
            pltpu.VMEM((CH,), jnp.int32),
            pltpu.VMEM((CH, H), _F32),
            pltpu.SemaphoreType.DMA,
        ],
    )(hs_cat, src, dst, zrows)


def _edge_body(a_hbm, b_hbm, src_hbm, tgt_hbm, mw2_hbm, mb2_hbm, out_hbm,
               si_v, ti_v, ar_v, br_v, mw2_v, mb2_v, ob_v, sem1, sem2):
    c = lax.axis_index("c")
    s = lax.axis_index("s")
    w = s * NC + c
    e_per_w = EP // (NC * NS)
    base_all = w * e_per_w
    pltpu.sync_copy(mw2_hbm, mw2_v)
    pltpu.sync_copy(mb2_hbm, mb2_v)
    mb2 = mb2_v[...]
    lane = lax.iota(jnp.int32, 16)

    def chunk(i, carry):
        base = base_all + i * CH
        pltpu.sync_copy(src_hbm.at[pl.ds(base, CH)], si_v)
        pltpu.sync_copy(tgt_hbm.at[pl.ds(base, CH)], ti_v)
        ca = pltpu.async_copy(a_hbm.at[si_v], ar_v, sem1)
        cb = pltpu.async_copy(b_hbm.at[ti_v], br_v, sem2)
        ca.wait()
        cb.wait()

        def grp(g8, carry2):
            eids = g8 * 16 + lane

            def ch16(g, acc):
                mg = mw2_v[pl.ds(g * 16, 16)]
                base_c = g * 16
                for k in range(16):
                    ci = jnp.full((16,), base_c + k, jnp.int32)
                    av = plsc.load_gather(ar_v, [eids, ci])
                    bv = plsc.load_gather(br_v, [eids, ci])
                    acc = acc + jnp.maximum(av + bv, 0.0) * mg[k]
                return acc

            acc = lax.fori_loop(0, C // 16, ch16, mb2)
            ob_v[pl.ds(g8 * 16, 16)] = acc
            return carry2

        lax.fori_loop(0, CH // 16, grp, 0)
        pltpu.sync_copy(ob_v, out_hbm.at[pl.ds(base, CH)])
        return carry

    lax.fori_loop(0, e_per_w // CH, chunk, 0)


def _edge(a, b, src, tgt, mw2, mb2_16):
    return pl.kernel(
        _edge_body,
        out_type=jax.ShapeDtypeStruct((EP,), _F32),
        mesh=_mesh(),
        compiler_params=pltpu.CompilerParams(needs_layout_passes=False),
        scratch_types=[
            pltpu.VMEM((CH,), jnp.int32),
            pltpu.VMEM((CH,), jnp.int32),
            pltpu.VMEM((CH, C), _F32),
            pltpu.VMEM((CH, C), _F32),
            pltpu.VMEM((C,), _F32),
            pltpu.VMEM((16,), _F32),
            pltpu.VMEM((CH,), _F32),
            pltpu.SemaphoreType.DMA,
            pltpu.SemaphoreType.DMA,
        ],
    )(a, b, src, tgt, mw2, mb2_16)


def _mm(a, b):
    return jax.lax.dot_general(a, b, (((1,), (0,)), ((), ())),
                               preferred_element_type=_F32, precision=_HIGH)


def _tc1_body(x_ref, w_ref, deg_ref, hs_ref, dinv_ref):
    dvec = jax.lax.rsqrt(deg_ref[:, 0:1] + deg_ref[:, 1:2] + 1.0)
    hs_ref[...] = _mm(x_ref[...], w_ref[...]) * dvec
    dinv_ref[...] = dvec


def _tc1(x_pad, W1, deg01):
    nb = NP // C
    return pl.pallas_call(
        _tc1_body,
        grid=(nb, 2),
        in_specs=[
            pl.BlockSpec((C, C), lambda i, h: (i, 0)),
            pl.BlockSpec((C, H), lambda i, h: (0, h)),
            pl.BlockSpec((C, 2), lambda i, h: (i, 0)),
        ],
        out_specs=[
            pl.BlockSpec((C, H), lambda i, h: (h * nb + i, 0)),
            pl.BlockSpec((C, 1), lambda i, h: (i, 0)),
        ],
        out_shape=[
            jax.ShapeDtypeStruct((NC * NP, H), _F32),
            jax.ShapeDtypeStruct((NP, 1), _F32),
        ],
    )(x_pad, W1, deg01)


def _tc2_body(accl, accr, hsl, hsr, dinv, b1, w2, o_ref):
    tot = jnp.concatenate([accl[...] + hsl[...], accr[...] + hsr[...]], axis=1)
    hfull = jnp.maximum(tot * dinv[...] + b1[0:1, :], 0.0)
    o_ref[...] = _mm(hfull, w2[...]) * dinv[...]


def _tc2(accL, accR, hsL, hsR, dinv, b1b, W2):
    nb = NP // C
    return pl.pallas_call(
        _tc2_body,
        grid=(nb, 2),
        in_specs=[
            pl.BlockSpec((C, H), lambda i, h: (i, 0)),
            pl.BlockSpec((C, H), lambda i, h: (i, 0)),
            pl.BlockSpec((C, H), lambda i, h: (i, 0)),
            pl.BlockSpec((C, H), lambda i, h: (i, 0)),
            pl.BlockSpec((C, 1), lambda i, h: (i, 0)),
            pl.BlockSpec((8, C), lambda i, h: (0, 0)),
            pl.BlockSpec((C, H), lambda i, h: (0, h)),
        ],
        out_specs=[pl.BlockSpec((C, H), lambda i, h: (h * nb + i, 0))],
        out_shape=[jax.ShapeDtypeStruct((NC * NP, H), _F32)],
    )(accL, accR, hsL, hsR, dinv, b1b, W2)[0]


def _tc3_body(accl, accr, hsl, hsr, dinv, b2, mw1, mb1, a_ref, b_ref):
    tot = jnp.concatenate([accl[...] + hsl[...], accr[...] + hsr[...]], axis=1)
    h2 = tot * dinv[...] + b2[0:1, :]
    a_ref[...] = _mm(h2, mw1[0:C, :])
    b_ref[...] = _mm(h2, mw1[C:2 * C, :]) + mb1[0:1, :]


def _tc3(accL, accR, hsL, hsR, dinv, b2b, MW1, mb1b):
    nb = NP // C
    return pl.pallas_call(
        _tc3_body,
        grid=(nb,),
        in_specs=[
            pl.BlockSpec((C, H), lambda i: (i, 0)),
            pl.BlockSpec((C, H), lambda i: (i, 0)),
            pl.BlockSpec((C, H), lambda i: (i, 0)),
            pl.BlockSpec((C, H), lambda i: (i, 0)),
            pl.BlockSpec((C, 1), lambda i: (i, 0)),
            pl.BlockSpec((8, C), lambda i: (0, 0)),
            pl.BlockSpec((2 * C, C), lambda i: (0, 0)),
            pl.BlockSpec((8, C), lambda i: (0, 0)),
        ],
        out_specs=[
            pl.BlockSpec((C, C), lambda i: (i, 0)),
            pl.BlockSpec((C, C), lambda i: (i, 0)),
        ],
        out_shape=[
            jax.ShapeDtypeStruct((NP, C), _F32),
            jax.ShapeDtypeStruct((NP, C), _F32),
        ],
    )(accL, accR, hsL, hsR, dinv, b2b, MW1, mb1b)


def kernel(x, edge_index, W1, b1, W2, b2, MW1, Mb1, MW2, Mb2):
    ei = edge_index.astype(jnp.int32)
    pad_idx = N + (jnp.arange(EP - E, dtype=jnp.int32) % (NP - N))
    src = jnp.concatenate([ei[0], pad_idx])
    dst = jnp.concatenate([ei[1], pad_idx])
    x_pad = jnp.concatenate([x, jnp.zeros((NP - N, C), _F32)], axis=0)
    zeros_t = jnp.zeros((ROWS_T,), _F32)
    ones_c = jnp.ones((CH,), _F32)
    zrows = jnp.zeros((CH, H), _F32)
    b1b = jnp.broadcast_to(b1[None, :], (8, C))
    b2b = jnp.broadcast_to(b2[None, :], (8, C))
    mb1b = jnp.broadcast_to(Mb1[None, :], (8, C))

    degp = _deg(dst, zeros_t, ones_c)
    deg01 = degp.reshape(NC, NP).transpose(1, 0)
    hs1, dinv = _tc1(x_pad, W1, deg01)
    acc1 = _conv(hs1, src, dst, zrows)
    hs2 = _tc2(acc1[:NP], acc1[NP:], hs1[:NP], hs1[NP:], dinv, b1b, W2)
    acc2 = _conv(hs2, src, dst, zrows)
    A, B = _tc3(acc2[:NP], acc2[NP:], hs2[:NP], hs2[NP:], dinv, b2b, MW1, mb1b)
    logits_pad = _edge(A, B, src, dst, MW2[:, 0],
                       jnp.broadcast_to(Mb2, (16,)))
    return logits_pad[:E]
